# baseline (speedup 1.0000x reference)
import functools

import jax
import jax.numpy as jnp
from jax import lax
from jax.experimental import pallas as pl
from jax.experimental.pallas import tpu as pltpu
from jax.experimental.pallas import tpu_sc as plsc

N = 10000
E = 320000
E2 = E + N
NIN = 128
NHID = 128
NOUT = 128
HEADS = 8
HC = HEADS * NHID

NC = 2
NS = 16
NW = NC * NS
SUB = 32
NSUB = 328
CHUNK = NSUB * SUB
EP = NW * CHUNK
NPAD = 10240

BN = 2000
ROWS_PER_TILE = NPAD // NS


def _tc1_body(x_ref, w1_ref, ps_ref, pd_ref, h_ref, as_ref, ad_ref,
              ms_ref, md_ref):
    hb = jnp.dot(x_ref[...], w1_ref[...], preferred_element_type=jnp.float32)
    h_ref[...] = hb.astype(jnp.bfloat16)
    s = jnp.dot(hb, ps_ref[...], preferred_element_type=jnp.float32)
    d = jnp.dot(hb, pd_ref[...], preferred_element_type=jnp.float32)
    as_ref[...] = s
    ad_ref[...] = d

    @pl.when(pl.program_id(0) == 0)
    def _():
        ms_ref[...] = jnp.full((8, 16), -1e30, jnp.float32)
        md_ref[...] = jnp.full((8, 16), -1e30, jnp.float32)

    ms_ref[...] = jnp.maximum(
        ms_ref[...], jnp.broadcast_to(jnp.max(s, axis=0, keepdims=True), (8, 16)))
    md_ref[...] = jnp.maximum(
        md_ref[...], jnp.broadcast_to(jnp.max(d, axis=0, keepdims=True), (8, 16)))


def _tc1(x, W1, Ps, Pd):
    nb = N // BN
    return pl.pallas_call(
        _tc1_body,
        grid=(nb,),
        in_specs=[
            pl.BlockSpec((BN, NIN), lambda b: (b, 0)),
            pl.BlockSpec((NIN, HC), lambda b: (0, 0)),
            pl.BlockSpec((HC, 16), lambda b: (0, 0)),
            pl.BlockSpec((HC, 16), lambda b: (0, 0)),
        ],
        out_specs=[
            pl.BlockSpec((BN, HC), lambda b: (b, 0)),
            pl.BlockSpec((BN, 16), lambda b: (b, 0)),
            pl.BlockSpec((BN, 16), lambda b: (b, 0)),
            pl.BlockSpec((8, 16), lambda b: (0, 0)),
            pl.BlockSpec((8, 16), lambda b: (0, 0)),
        ],
        out_shape=[
            jax.ShapeDtypeStruct((N, HC), jnp.bfloat16),
            jax.ShapeDtypeStruct((N, 16), jnp.float32),
            jax.ShapeDtypeStruct((N, 16), jnp.float32),
            jax.ShapeDtypeStruct((8, 16), jnp.float32),
            jax.ShapeDtypeStruct((8, 16), jnp.float32),
        ],
    )(x, W1, Ps, Pd)


def _sc1_compute(svX, dvX, wvX, mv, gbase):
    def edge(i, c2):
        e = svX[i, :] + dvX[i, :]
        e = jnp.maximum(e, 0.2 * e)
        w = jnp.exp(e - mv[...])
        valid = (gbase + i) < E2
        wvX[i, :] = jnp.where(valid, w, 0.0)
        return c2
    lax.fori_loop(0, SUB, edge, 0)


def _sc1_body(asrc_hbm, adst_hbm, src_hbm, dst_hbm, m_hbm, z16_hbm,
              w_hbm, d0_hbm, d1_hbm,
              src_v, dst_v, svA, svB, dvA, dvB, wvA, wvB, mv, tmp, dsh,
              semGA, semGB, semWA, semWB, semSA, semSB):
    cid = lax.axis_index("c")
    sid = lax.axis_index("s")
    wid = sid * NC + cid
    rowbase = wid * NSUB

    @pl.when(sid == 0)
    def _():
        pltpu.sync_copy(z16_hbm, dsh)
    plsc.subcore_barrier()

    pltpu.sync_copy(src_hbm.at[pl.ds(rowbase, NSUB)], src_v)
    pltpu.sync_copy(dst_hbm.at[pl.ds(rowbase, NSUB)], dst_v)
    pltpu.sync_copy(m_hbm, mv)

    def gathers(r, svX, dvX, sem):
        return [
            pltpu.make_async_copy(asrc_hbm.at[src_v.at[r]], svX, sem),
            pltpu.make_async_copy(adst_hbm.at[dst_v.at[r]], dvX, sem),
        ]

    def half(bp, r, svX, dvX, wvX, semG, semW, semS):
        gbase = (rowbase + r) * SUB

        @pl.when(bp >= 1)
        def _():
            pltpu.make_async_copy(
                wvX, w_hbm.at[pl.ds(gbase, SUB)], semW).wait()
            pltpu.make_async_copy(wvX, dsh.at[dst_v.at[r]], semS).wait()
        _sc1_compute(svX, dvX, wvX, mv, gbase)
        pltpu.async_copy(wvX, w_hbm.at[pl.ds(gbase, SUB)], semW)
        pltpu.async_copy(wvX, dsh.at[dst_v.at[r]], semS, add=True)

    for c in gathers(0, svA, dvA, semGA):
        c.start()

    def pair(bp, carry):
        rA = 2 * bp
        rB = rA + 1
        for c in gathers(rB, svB, dvB, semGB):
            c.start()
        for c in gathers(rA, svA, dvA, semGA):
            c.wait()
        half(bp, rA, svA, dvA, wvA, semGA, semWA, semSA)

        @pl.when(bp < NSUB // 2 - 1)
        def _():
            for c in gathers(rA + 2, svA, dvA, semGA):
                c.start()
        for c in gathers(rB, svB, dvB, semGB):
            c.wait()
        half(bp, rB, svB, dvB, wvB, semGB, semWB, semSB)
        return carry
    lax.fori_loop(0, NSUB // 2, pair, 0)

    pltpu.make_async_copy(
        wvA, w_hbm.at[pl.ds(rowbase * SUB, SUB)], semWA).wait()
    pltpu.make_async_copy(wvA, dsh.at[dst_v.at[0]], semSA).wait()
    pltpu.make_async_copy(
        wvB, w_hbm.at[pl.ds(rowbase * SUB, SUB)], semWB).wait()
    pltpu.make_async_copy(wvB, dsh.at[dst_v.at[0]], semSB).wait()

    plsc.subcore_barrier()
    r0 = sid * ROWS_PER_TILE
    pltpu.sync_copy(dsh.at[pl.ds(r0, ROWS_PER_TILE)], tmp)

    @pl.when(cid == 0)
    def _():
        pltpu.sync_copy(tmp, d0_hbm.at[pl.ds(r0, ROWS_PER_TILE)])

    @pl.when(cid == 1)
    def _():
        pltpu.sync_copy(tmp, d1_hbm.at[pl.ds(r0, ROWS_PER_TILE)])


def _sc1(asrc16, adst16, src2d, dst2d, m16, z16):
    mesh = plsc.VectorSubcoreMesh(core_axis_name="c", subcore_axis_name="s")
    f = pl.kernel(
        _sc1_body,
        out_type=[
            jax.ShapeDtypeStruct((EP, 16), jnp.float32),
            jax.ShapeDtypeStruct((NPAD, 16), jnp.float32),
            jax.ShapeDtypeStruct((NPAD, 16), jnp.float32),
        ],
        mesh=mesh,
        scratch_types=[
            pltpu.VMEM((NSUB, SUB), jnp.int32),
            pltpu.VMEM((NSUB, SUB), jnp.int32),
            pltpu.VMEM((SUB, 16), jnp.float32),
            pltpu.VMEM((SUB, 16), jnp.float32),
            pltpu.VMEM((SUB, 16), jnp.float32),
            pltpu.VMEM((SUB, 16), jnp.float32),
            pltpu.VMEM((SUB, 16), jnp.float32),
            pltpu.VMEM((SUB, 16), jnp.float32),
            pltpu.VMEM((16,), jnp.float32),
            pltpu.VMEM((ROWS_PER_TILE, 16), jnp.float32),
            pltpu.VMEM_SHARED((NPAD, 16), jnp.float32),
            pltpu.SemaphoreType.DMA,
            pltpu.SemaphoreType.DMA,
            pltpu.SemaphoreType.DMA,
            pltpu.SemaphoreType.DMA,
            pltpu.SemaphoreType.DMA,
            pltpu.SemaphoreType.DMA,
        ],
        compiler_params=pltpu.CompilerParams(use_tc_tiling_on_sc=False),
    )
    return f(asrc16, adst16, src2d, dst2d, m16, z16)


def _compute_block(hvX, wvX, d0X, d1X, mgvX):
    def edge(i, c2):
        den = d0X[i, :] + d1X[i, :]
        alpha = wvX[i, :] / (den + 1e-16)
        acc = [None] * (NHID // 16)
        for h in range(HEADS):
            a = alpha[h]
            for g in range(NHID // 32):
                v = hvX[i, pl.ds(h * (NHID // 2) + g * 16, 16)]
                lo = plsc.bitcast(v << 16, jnp.float32)
                hi = plsc.bitcast(v & jnp.int32(-65536), jnp.float32)
                tl = a * lo
                th = a * hi
                acc[2 * g] = tl if acc[2 * g] is None else acc[2 * g] + tl
                acc[2 * g + 1] = th if acc[2 * g + 1] is None else acc[2 * g + 1] + th
        for cb in range(NHID // 16):
            mgvX[i, pl.ds(cb * 16, 16)] = acc[cb]
        return c2
    lax.fori_loop(0, 16, edge, 0)


def _sc2_body(h_hbm, w_hbm, d0_hbm, d1_hbm, src_hbm, dst_hbm, z128_hbm,
              o0_hbm, o1_hbm,
              src_pg, dst_pg, hvA, hvB, wvA, wvB, d0A, d0B, d1A, d1B,
              mgvA, mgvB, ringA, ringB, tmp, osh,
              semGA, semGB, semSA, semSB):
    cid = lax.axis_index("c")
    sid = lax.axis_index("s")
    wid = sid * NC + cid
    rowbase16 = wid * (CHUNK // 16)
    wbase = wid * CHUNK
    npairs = CHUNK // 32
    npages = CHUNK // 128

    @pl.when(sid == 0)
    def _():
        pltpu.sync_copy(z128_hbm, osh)
    plsc.subcore_barrier()

    def gathers(b, r, hvX, wvX, d0X, d1X, sem):
        return [
            pltpu.make_async_copy(h_hbm.at[src_pg.at[r]], hvX, sem),
            pltpu.make_async_copy(d0_hbm.at[dst_pg.at[r]], d0X, sem),
            pltpu.make_async_copy(d1_hbm.at[dst_pg.at[r]], d1X, sem),
            pltpu.make_async_copy(
                w_hbm.at[pl.ds(wbase + b * 16, 16)], wvX, sem),
        ]

    def issue(b, r, hvX, wvX, d0X, d1X, sem):
        for c in gathers(b, r, hvX, wvX, d0X, d1X, sem):
            c.start()

    def drain(b, r, hvX, wvX, d0X, d1X, sem):
        for c in gathers(b, r, hvX, wvX, d0X, d1X, sem):
            c.wait()

    pltpu.sync_copy(src_hbm.at[pl.ds(rowbase16, 8)], src_pg.at[pl.ds(0, 8)])
    pltpu.sync_copy(dst_hbm.at[pl.ds(rowbase16, 8)], dst_pg.at[pl.ds(0, 8)])
    issue(0, 0, hvA, wvA, d0A, d1A, semGA)

    def pair(bp, carry):
        bA = 2 * bp
        bB = bA + 1
        rA = bA & 15
        rB = bB & 15

        @pl.when(((bp & 3) == 0) & (bp < npairs - 4))
        def _():
            pgn = (bp >> 2) + 1
            off = (pgn & 1) * 8
            pltpu.sync_copy(src_hbm.at[pl.ds(rowbase16 + pgn * 8, 8)],
                            src_pg.at[pl.ds(off, 8)])
            pltpu.sync_copy(dst_hbm.at[pl.ds(rowbase16 + pgn * 8, 8)],
                            dst_pg.at[pl.ds(off, 8)])

        issue(bB, rB, hvB, wvB, d0B, d1B, semGB)
        drain(bA, rA, hvA, wvA, d0A, d1A, semGA)

        @pl.when(bp >= 1)
        def _():
            pltpu.make_async_copy(mgvA, osh.at[ringA.at[0]], semSA).wait()
        _compute_block(hvA, wvA, d0A, d1A, mgvA)
        ringA[0, :] = dst_pg[rA, :]
        pltpu.async_copy(mgvA, osh.at[ringA.at[0]], semSA, add=True)

        @pl.when(bp < npairs - 1)
        def _():
            b2 = bA + 2
            issue(b2, b2 & 15, hvA, wvA, d0A, d1A, semGA)
        drain(bB, rB, hvB, wvB, d0B, d1B, semGB)

        @pl.when(bp >= 1)
        def _():
            pltpu.make_async_copy(mgvB, osh.at[ringB.at[0]], semSB).wait()
        _compute_block(hvB, wvB, d0B, d1B, mgvB)
        ringB[0, :] = dst_pg[rB, :]
        pltpu.async_copy(mgvB, osh.at[ringB.at[0]], semSB, add=True)
        return carry
    lax.fori_loop(0, npairs, pair, 0)

    pltpu.make_async_copy(mgvA, osh.at[ringA.at[0]], semSA).wait()
    pltpu.make_async_copy(mgvB, osh.at[ringB.at[0]], semSB).wait()

    plsc.subcore_barrier()
    for q in range(ROWS_PER_TILE // 32):
        r0 = sid * ROWS_PER_TILE + q * 32
        pltpu.sync_copy(osh.at[pl.ds(r0, 32)], tmp)

        @pl.when(cid == 0)
        def _():
            pltpu.sync_copy(tmp, o0_hbm.at[pl.ds(r0, 32)])

        @pl.when(cid == 1)
        def _():
            pltpu.sync_copy(tmp, o1_hbm.at[pl.ds(r0, 32)])


def _sc2(h, w, d0, d1, src16, dst16, z128):
    mesh = plsc.VectorSubcoreMesh(core_axis_name="c", subcore_axis_name="s")
    f = pl.kernel(
        _sc2_body,
        out_type=[
            jax.ShapeDtypeStruct((NPAD, NHID), jnp.float32),
            jax.ShapeDtypeStruct((NPAD, NHID), jnp.float32),
        ],
        mesh=mesh,
        scratch_types=[
            pltpu.VMEM((16, 16), jnp.int32),
            pltpu.VMEM((16, 16), jnp.int32),
            pltpu.VMEM((16, HC // 2), jnp.int32),
            pltpu.VMEM((16, HC // 2), jnp.int32),
            pltpu.VMEM((16, 16), jnp.float32),
            pltpu.VMEM((16, 16), jnp.float32),
            pltpu.VMEM((16, 16), jnp.float32),
            pltpu.VMEM((16, 16), jnp.float32),
            pltpu.VMEM((16, 16), jnp.float32),
            pltpu.VMEM((16, 16), jnp.float32),
            pltpu.VMEM((16, NHID), jnp.float32),
            pltpu.VMEM((16, NHID), jnp.float32),
            pltpu.VMEM((8, 16), jnp.int32),
            pltpu.VMEM((8, 16), jnp.int32),
            pltpu.VMEM((32, NHID), jnp.float32),
            pltpu.VMEM_SHARED((NPAD, NHID), jnp.float32),
            pltpu.SemaphoreType.DMA,
            pltpu.SemaphoreType.DMA,
            pltpu.SemaphoreType.DMA,
            pltpu.SemaphoreType.DMA,
        ],
        compiler_params=pltpu.CompilerParams(
            use_tc_tiling_on_sc=False, needs_layout_passes=False),
    )
    return f(h, w, d0, d1, src16, dst16, z128)


def _tc2_body(o0_ref, o1_ref, pm_ref, b1_ref, wd_ref, bd_ref, out_ref, h_ref):
    operm = (o0_ref[...] + o1_ref[...]) * (1.0 / HEADS)
    o = jnp.dot(operm, pm_ref[...], preferred_element_type=jnp.float32) \
        + b1_ref[...]
    hh = jnp.where(o > 0, o, jnp.exp(jnp.minimum(o, 0.0)) - 1.0)
    h_ref[...] = hh
    out_ref[...] = (
        jnp.dot(hh, wd_ref[...], preferred_element_type=jnp.float32)
        + bd_ref[...]
    )


def _tc2(o0, o1, Pm, b1, Wd, bd):
    nb = N // BN
    return pl.pallas_call(
        _tc2_body,
        grid=(nb,),
        in_specs=[
            pl.BlockSpec((BN, NHID), lambda b: (b, 0), ),
            pl.BlockSpec((BN, NHID), lambda b: (b, 0)),
            pl.BlockSpec((NHID, NHID), lambda b: (0, 0)),
            pl.BlockSpec((1, NHID), lambda b: (0, 0)),
            pl.BlockSpec((NHID, NOUT), lambda b: (0, 0)),
            pl.BlockSpec((1, NOUT), lambda b: (0, 0)),
        ],
        out_specs=[
            pl.BlockSpec((BN, NOUT), lambda b: (b, 0)),
            pl.BlockSpec((BN, NHID), lambda b: (b, 0)),
        ],
        out_shape=[
            jax.ShapeDtypeStruct((N, NOUT), jnp.float32),
            jax.ShapeDtypeStruct((N, NHID), jnp.float32),
        ],
    )(o0, o1, Pm, b1.reshape(1, NHID), Wd, bd.reshape(1, NOUT))


def kernel(x, edge_index, W1, att_src, att_dst, b1, Wd, bd):
    eye8 = jnp.eye(HEADS, dtype=jnp.float32)
    ps = (att_src[0][:, :, None] * eye8[:, None, :])
    pd = (att_dst[0][:, :, None] * eye8[:, None, :])
    Ps = jnp.concatenate([ps, ps], axis=-1).reshape(HC, 16)
    Pd = jnp.concatenate([pd, pd], axis=-1).reshape(HC, 16)

    loop = jnp.arange(N, dtype=edge_index.dtype)
    pad = jnp.zeros((EP - E2,), dtype=edge_index.dtype)
    src2d = jnp.concatenate([edge_index[0], loop, pad]).reshape(NW * NSUB, SUB)
    dst2d = jnp.concatenate([edge_index[1], loop, pad]).reshape(NW * NSUB, SUB)

    h, asrc16, adst16, ms, md = _tc1(x, W1, Ps, Pd)
    m16 = jnp.maximum(ms.max(axis=0) + md.max(axis=0), 0.0)

    z16 = jnp.zeros((NPAD, 16), jnp.float32)
    z128 = jnp.zeros((NPAD, NHID), jnp.float32)

    w, d0, d1 = _sc1(asrc16, adst16, src2d, dst2d, m16, z16)
    h = jax.lax.bitcast_convert_type(
        h.reshape(N, HC // 2, 2), jnp.int32)
    src16 = src2d.reshape(EP // 16, 16)
    dst16 = dst2d.reshape(EP // 16, 16)
    o0, o1 = _sc2(h, w, d0, d1, src16, dst16, z128)

    g = jnp.arange(NHID // 32)[:, None]
    j = jnp.arange(16)[None, :]
    tchan = jnp.concatenate(
        [g * 32 + 2 * j, g * 32 + 2 * j + 1], axis=1).reshape(NHID)
    Pm = jnp.zeros((NHID, NHID), jnp.float32).at[
        jnp.arange(NHID), tchan].set(1.0)

    out, hstate = _tc2(o0[:N], o1[:N], Pm, b1, Wd, bd)
    return (out, hstate)


# speedup vs baseline: 1.3148x; 1.3148x over previous
import functools

import jax
import jax.numpy as jnp
from jax import lax
from jax.experimental import pallas as pl
from jax.experimental.pallas import tpu as pltpu
from jax.experimental.pallas import tpu_sc as plsc

N = 10000
E = 320000
E2 = E + N
NIN = 128
NHID = 128
NOUT = 128
HEADS = 8
HC = HEADS * NHID

NC = 2
NS = 16
NW = NC * NS
SUB = 32
NSUB = 328
CHUNK = NSUB * SUB
EP = NW * CHUNK
NPAD = 10240

BN = 2000
ROWS_PER_TILE = NPAD // NS


def _tc1_body(x_ref, w1_ref, ps_ref, pd_ref, h_ref, as_ref, ad_ref,
              ms_ref, md_ref):
    hb = jnp.dot(x_ref[...], w1_ref[...], preferred_element_type=jnp.float32)
    h_ref[...] = hb.astype(jnp.bfloat16)
    s = jnp.dot(hb, ps_ref[...], preferred_element_type=jnp.float32)
    d = jnp.dot(hb, pd_ref[...], preferred_element_type=jnp.float32)
    as_ref[...] = s
    ad_ref[...] = d

    @pl.when(pl.program_id(0) == 0)
    def _():
        ms_ref[...] = jnp.full((8, 16), -1e30, jnp.float32)
        md_ref[...] = jnp.full((8, 16), -1e30, jnp.float32)

    ms_ref[...] = jnp.maximum(
        ms_ref[...], jnp.broadcast_to(jnp.max(s, axis=0, keepdims=True), (8, 16)))
    md_ref[...] = jnp.maximum(
        md_ref[...], jnp.broadcast_to(jnp.max(d, axis=0, keepdims=True), (8, 16)))


def _tc1(x, W1, Ps, Pd):
    nb = N // BN
    return pl.pallas_call(
        _tc1_body,
        grid=(nb,),
        in_specs=[
            pl.BlockSpec((BN, NIN), lambda b: (b, 0)),
            pl.BlockSpec((NIN, HC), lambda b: (0, 0)),
            pl.BlockSpec((HC, 16), lambda b: (0, 0)),
            pl.BlockSpec((HC, 16), lambda b: (0, 0)),
        ],
        out_specs=[
            pl.BlockSpec((BN, HC), lambda b: (b, 0)),
            pl.BlockSpec((BN, 16), lambda b: (b, 0)),
            pl.BlockSpec((BN, 16), lambda b: (b, 0)),
            pl.BlockSpec((8, 16), lambda b: (0, 0)),
            pl.BlockSpec((8, 16), lambda b: (0, 0)),
        ],
        out_shape=[
            jax.ShapeDtypeStruct((N, HC), jnp.bfloat16),
            jax.ShapeDtypeStruct((N, 16), jnp.float32),
            jax.ShapeDtypeStruct((N, 16), jnp.float32),
            jax.ShapeDtypeStruct((8, 16), jnp.float32),
            jax.ShapeDtypeStruct((8, 16), jnp.float32),
        ],
    )(x, W1, Ps, Pd)


def _sc1_compute(svX, dvX, wvX, mv, gbase):
    def edge(i, c2):
        e = svX[i, :] + dvX[i, :]
        e = jnp.maximum(e, 0.2 * e)
        w = jnp.exp(e - mv[...])
        valid = (gbase + i) < E2
        wvX[i, :] = jnp.where(valid, w, 0.0)
        return c2
    lax.fori_loop(0, SUB, edge, 0)


def _sc1_body(asrc_hbm, adst_hbm, src_hbm, dst_hbm, m_hbm, z16_hbm,
              w_hbm, d0_hbm, d1_hbm,
              src_v, dst_v, svA, svB, dvA, dvB, wvA, wvB, mv, tmp, dsh,
              semGA, semGB, semWA, semWB, semSA, semSB):
    cid = lax.axis_index("c")
    sid = lax.axis_index("s")
    wid = sid * NC + cid
    rowbase = wid * NSUB

    @pl.when(sid == 0)
    def _():
        pltpu.sync_copy(z16_hbm, dsh)
    plsc.subcore_barrier()

    pltpu.sync_copy(src_hbm.at[pl.ds(rowbase, NSUB)], src_v)
    pltpu.sync_copy(dst_hbm.at[pl.ds(rowbase, NSUB)], dst_v)
    pltpu.sync_copy(m_hbm, mv)

    def gathers(r, svX, dvX, sem):
        return [
            pltpu.make_async_copy(asrc_hbm.at[src_v.at[r]], svX, sem),
            pltpu.make_async_copy(adst_hbm.at[dst_v.at[r]], dvX, sem),
        ]

    def half(bp, r, svX, dvX, wvX, semG, semW, semS):
        gbase = (rowbase + r) * SUB

        @pl.when(bp >= 1)
        def _():
            pltpu.make_async_copy(
                wvX, w_hbm.at[pl.ds(gbase, SUB)], semW).wait()
            pltpu.make_async_copy(wvX, dsh.at[dst_v.at[r]], semS).wait()
        _sc1_compute(svX, dvX, wvX, mv, gbase)
        pltpu.async_copy(wvX, w_hbm.at[pl.ds(gbase, SUB)], semW)
        pltpu.async_copy(wvX, dsh.at[dst_v.at[r]], semS, add=True)

    for c in gathers(0, svA, dvA, semGA):
        c.start()

    def pair(bp, carry):
        rA = 2 * bp
        rB = rA + 1
        for c in gathers(rB, svB, dvB, semGB):
            c.start()
        for c in gathers(rA, svA, dvA, semGA):
            c.wait()
        half(bp, rA, svA, dvA, wvA, semGA, semWA, semSA)

        @pl.when(bp < NSUB // 2 - 1)
        def _():
            for c in gathers(rA + 2, svA, dvA, semGA):
                c.start()
        for c in gathers(rB, svB, dvB, semGB):
            c.wait()
        half(bp, rB, svB, dvB, wvB, semGB, semWB, semSB)
        return carry
    lax.fori_loop(0, NSUB // 2, pair, 0)

    pltpu.make_async_copy(
        wvA, w_hbm.at[pl.ds(rowbase * SUB, SUB)], semWA).wait()
    pltpu.make_async_copy(wvA, dsh.at[dst_v.at[0]], semSA).wait()
    pltpu.make_async_copy(
        wvB, w_hbm.at[pl.ds(rowbase * SUB, SUB)], semWB).wait()
    pltpu.make_async_copy(wvB, dsh.at[dst_v.at[0]], semSB).wait()

    plsc.subcore_barrier()
    r0 = sid * ROWS_PER_TILE
    pltpu.sync_copy(dsh.at[pl.ds(r0, ROWS_PER_TILE)], tmp)

    @pl.when(cid == 0)
    def _():
        pltpu.sync_copy(tmp, d0_hbm.at[pl.ds(r0, ROWS_PER_TILE)])

    @pl.when(cid == 1)
    def _():
        pltpu.sync_copy(tmp, d1_hbm.at[pl.ds(r0, ROWS_PER_TILE)])


def _sc1(asrc16, adst16, src2d, dst2d, m16, z16):
    mesh = plsc.VectorSubcoreMesh(core_axis_name="c", subcore_axis_name="s")
    f = pl.kernel(
        _sc1_body,
        out_type=[
            jax.ShapeDtypeStruct((EP, 16), jnp.float32),
            jax.ShapeDtypeStruct((NPAD, 16), jnp.float32),
            jax.ShapeDtypeStruct((NPAD, 16), jnp.float32),
        ],
        mesh=mesh,
        scratch_types=[
            pltpu.VMEM((NSUB, SUB), jnp.int32),
            pltpu.VMEM((NSUB, SUB), jnp.int32),
            pltpu.VMEM((SUB, 16), jnp.float32),
            pltpu.VMEM((SUB, 16), jnp.float32),
            pltpu.VMEM((SUB, 16), jnp.float32),
            pltpu.VMEM((SUB, 16), jnp.float32),
            pltpu.VMEM((SUB, 16), jnp.float32),
            pltpu.VMEM((SUB, 16), jnp.float32),
            pltpu.VMEM((16,), jnp.float32),
            pltpu.VMEM((ROWS_PER_TILE, 16), jnp.float32),
            pltpu.VMEM_SHARED((NPAD, 16), jnp.float32),
            pltpu.SemaphoreType.DMA,
            pltpu.SemaphoreType.DMA,
            pltpu.SemaphoreType.DMA,
            pltpu.SemaphoreType.DMA,
            pltpu.SemaphoreType.DMA,
            pltpu.SemaphoreType.DMA,
        ],
        compiler_params=pltpu.CompilerParams(use_tc_tiling_on_sc=False),
    )
    return f(asrc16, adst16, src2d, dst2d, m16, z16)


def _compute_block(hvX, wvX, d0X, d1X, mgvX):
    def edge(i, c2):
        den = d0X[i, :] + d1X[i, :]
        alpha = wvX[i, :] / (den + 1e-16)
        acc = [None] * (NHID // 16)
        for h in range(HEADS):
            a = alpha[h]
            for g in range(NHID // 32):
                hb32 = hvX[i, pl.ds(h * NHID + g * 32, 32)]
                lo, hi = plsc.unpack(hb32, format=plsc.PackFormat.INTERLEAVED)
                tl = a * lo
                th = a * hi
                acc[2 * g] = tl if acc[2 * g] is None else acc[2 * g] + tl
                acc[2 * g + 1] = th if acc[2 * g + 1] is None else acc[2 * g + 1] + th
        for cb in range(NHID // 16):
            mgvX[i, pl.ds(cb * 16, 16)] = acc[cb]
        return c2
    lax.fori_loop(0, 16, edge, 0)


def _sc2_body(h_hbm, w_hbm, d0_hbm, d1_hbm, src_hbm, dst_hbm, z128_hbm,
              o0_hbm, o1_hbm,
              src_pg, dst_pg, hvA, hvB, wvA, wvB, d0A, d0B, d1A, d1B,
              mgvA, mgvB, ringA, ringB, tmp, osh,
              semGA, semGB, semSA, semSB):
    cid = lax.axis_index("c")
    sid = lax.axis_index("s")
    wid = sid * NC + cid
    rowbase16 = wid * (CHUNK // 16)
    wbase = wid * CHUNK
    npairs = CHUNK // 32
    npages = CHUNK // 128

    @pl.when(sid == 0)
    def _():
        pltpu.sync_copy(z128_hbm, osh)
    plsc.subcore_barrier()

    def gathers(b, r, hvX, wvX, d0X, d1X, sem):
        return [
            pltpu.make_async_copy(h_hbm.at[src_pg.at[r]], hvX, sem),
            pltpu.make_async_copy(d0_hbm.at[dst_pg.at[r]], d0X, sem),
            pltpu.make_async_copy(d1_hbm.at[dst_pg.at[r]], d1X, sem),
            pltpu.make_async_copy(
                w_hbm.at[pl.ds(wbase + b * 16, 16)], wvX, sem),
        ]

    def issue(b, r, hvX, wvX, d0X, d1X, sem):
        for c in gathers(b, r, hvX, wvX, d0X, d1X, sem):
            c.start()

    def drain(b, r, hvX, wvX, d0X, d1X, sem):
        for c in gathers(b, r, hvX, wvX, d0X, d1X, sem):
            c.wait()

    pltpu.sync_copy(src_hbm.at[pl.ds(rowbase16, 8)], src_pg.at[pl.ds(0, 8)])
    pltpu.sync_copy(dst_hbm.at[pl.ds(rowbase16, 8)], dst_pg.at[pl.ds(0, 8)])
    issue(0, 0, hvA, wvA, d0A, d1A, semGA)

    def pair(bp, carry):
        bA = 2 * bp
        bB = bA + 1
        rA = bA & 15
        rB = bB & 15

        @pl.when(((bp & 3) == 0) & (bp < npairs - 4))
        def _():
            pgn = (bp >> 2) + 1
            off = (pgn & 1) * 8
            pltpu.sync_copy(src_hbm.at[pl.ds(rowbase16 + pgn * 8, 8)],
                            src_pg.at[pl.ds(off, 8)])
            pltpu.sync_copy(dst_hbm.at[pl.ds(rowbase16 + pgn * 8, 8)],
                            dst_pg.at[pl.ds(off, 8)])

        issue(bB, rB, hvB, wvB, d0B, d1B, semGB)
        drain(bA, rA, hvA, wvA, d0A, d1A, semGA)

        @pl.when(bp >= 1)
        def _():
            pltpu.make_async_copy(mgvA, osh.at[ringA.at[0]], semSA).wait()
        _compute_block(hvA, wvA, d0A, d1A, mgvA)
        ringA[0, :] = dst_pg[rA, :]
        pltpu.async_copy(mgvA, osh.at[ringA.at[0]], semSA, add=True)

        @pl.when(bp < npairs - 1)
        def _():
            b2 = bA + 2
            issue(b2, b2 & 15, hvA, wvA, d0A, d1A, semGA)
        drain(bB, rB, hvB, wvB, d0B, d1B, semGB)

        @pl.when(bp >= 1)
        def _():
            pltpu.make_async_copy(mgvB, osh.at[ringB.at[0]], semSB).wait()
        _compute_block(hvB, wvB, d0B, d1B, mgvB)
        ringB[0, :] = dst_pg[rB, :]
        pltpu.async_copy(mgvB, osh.at[ringB.at[0]], semSB, add=True)
        return carry
    lax.fori_loop(0, npairs, pair, 0)

    pltpu.make_async_copy(mgvA, osh.at[ringA.at[0]], semSA).wait()
    pltpu.make_async_copy(mgvB, osh.at[ringB.at[0]], semSB).wait()

    plsc.subcore_barrier()
    for q in range(ROWS_PER_TILE // 32):
        r0 = sid * ROWS_PER_TILE + q * 32
        pltpu.sync_copy(osh.at[pl.ds(r0, 32)], tmp)

        @pl.when(cid == 0)
        def _():
            pltpu.sync_copy(tmp, o0_hbm.at[pl.ds(r0, 32)])

        @pl.when(cid == 1)
        def _():
            pltpu.sync_copy(tmp, o1_hbm.at[pl.ds(r0, 32)])


def _sc2(h, w, d0, d1, src16, dst16, z128):
    mesh = plsc.VectorSubcoreMesh(core_axis_name="c", subcore_axis_name="s")
    f = pl.kernel(
        _sc2_body,
        out_type=[
            jax.ShapeDtypeStruct((NPAD, NHID), jnp.float32),
            jax.ShapeDtypeStruct((NPAD, NHID), jnp.float32),
        ],
        mesh=mesh,
        scratch_types=[
            pltpu.VMEM((16, 16), jnp.int32),
            pltpu.VMEM((16, 16), jnp.int32),
            pltpu.VMEM((16, HC), jnp.bfloat16),
            pltpu.VMEM((16, HC), jnp.bfloat16),
            pltpu.VMEM((16, 16), jnp.float32),
            pltpu.VMEM((16, 16), jnp.float32),
            pltpu.VMEM((16, 16), jnp.float32),
            pltpu.VMEM((16, 16), jnp.float32),
            pltpu.VMEM((16, 16), jnp.float32),
            pltpu.VMEM((16, 16), jnp.float32),
            pltpu.VMEM((16, NHID), jnp.float32),
            pltpu.VMEM((16, NHID), jnp.float32),
            pltpu.VMEM((8, 16), jnp.int32),
            pltpu.VMEM((8, 16), jnp.int32),
            pltpu.VMEM((32, NHID), jnp.float32),
            pltpu.VMEM_SHARED((NPAD, NHID), jnp.float32),
            pltpu.SemaphoreType.DMA,
            pltpu.SemaphoreType.DMA,
            pltpu.SemaphoreType.DMA,
            pltpu.SemaphoreType.DMA,
        ],
        compiler_params=pltpu.CompilerParams(
            use_tc_tiling_on_sc=False, needs_layout_passes=False),
    )
    return f(h, w, d0, d1, src16, dst16, z128)


def _tc2_body(o0_ref, o1_ref, pm_ref, b1_ref, wd_ref, bd_ref, out_ref, h_ref):
    operm = (o0_ref[...] + o1_ref[...]) * (1.0 / HEADS)
    o = jnp.dot(operm, pm_ref[...], preferred_element_type=jnp.float32) \
        + b1_ref[...]
    hh = jnp.where(o > 0, o, jnp.exp(jnp.minimum(o, 0.0)) - 1.0)
    h_ref[...] = hh
    out_ref[...] = (
        jnp.dot(hh, wd_ref[...], preferred_element_type=jnp.float32)
        + bd_ref[...]
    )


def _tc2(o0, o1, Pm, b1, Wd, bd):
    nb = N // BN
    return pl.pallas_call(
        _tc2_body,
        grid=(nb,),
        in_specs=[
            pl.BlockSpec((BN, NHID), lambda b: (b, 0), ),
            pl.BlockSpec((BN, NHID), lambda b: (b, 0)),
            pl.BlockSpec((NHID, NHID), lambda b: (0, 0)),
            pl.BlockSpec((1, NHID), lambda b: (0, 0)),
            pl.BlockSpec((NHID, NOUT), lambda b: (0, 0)),
            pl.BlockSpec((1, NOUT), lambda b: (0, 0)),
        ],
        out_specs=[
            pl.BlockSpec((BN, NOUT), lambda b: (b, 0)),
            pl.BlockSpec((BN, NHID), lambda b: (b, 0)),
        ],
        out_shape=[
            jax.ShapeDtypeStruct((N, NOUT), jnp.float32),
            jax.ShapeDtypeStruct((N, NHID), jnp.float32),
        ],
    )(o0, o1, Pm, b1.reshape(1, NHID), Wd, bd.reshape(1, NOUT))


def kernel(x, edge_index, W1, att_src, att_dst, b1, Wd, bd):
    eye8 = jnp.eye(HEADS, dtype=jnp.float32)
    ps = (att_src[0][:, :, None] * eye8[:, None, :])
    pd = (att_dst[0][:, :, None] * eye8[:, None, :])
    Ps = jnp.concatenate([ps, ps], axis=-1).reshape(HC, 16)
    Pd = jnp.concatenate([pd, pd], axis=-1).reshape(HC, 16)

    loop = jnp.arange(N, dtype=edge_index.dtype)
    pad = jnp.zeros((EP - E2,), dtype=edge_index.dtype)
    src2d = jnp.concatenate([edge_index[0], loop, pad]).reshape(NW * NSUB, SUB)
    dst2d = jnp.concatenate([edge_index[1], loop, pad]).reshape(NW * NSUB, SUB)

    h, asrc16, adst16, ms, md = _tc1(x, W1, Ps, Pd)
    m16 = jnp.maximum(ms.max(axis=0) + md.max(axis=0), 0.0)

    z16 = jnp.zeros((NPAD, 16), jnp.float32)
    z128 = jnp.zeros((NPAD, NHID), jnp.float32)

    w, d0, d1 = _sc1(asrc16, adst16, src2d, dst2d, m16, z16)
    src16 = src2d.reshape(EP // 16, 16)
    dst16 = dst2d.reshape(EP // 16, 16)
    o0, o1 = _sc2(h, w, d0, d1, src16, dst16, z128)

    g = jnp.arange(NHID // 32)[:, None]
    j = jnp.arange(16)[None, :]
    tchan = jnp.concatenate(
        [g * 32 + 2 * j, g * 32 + 2 * j + 1], axis=1).reshape(NHID)
    Pm = jnp.zeros((NHID, NHID), jnp.float32).at[
        jnp.arange(NHID), tchan].set(1.0)

    out, hstate = _tc2(o0[:N], o1[:N], Pm, b1, Wd, bd)
    return (out, hstate)
